# Initial kernel scaffold; baseline (speedup 1.0000x reference)
import jax
import jax.numpy as jnp
from jax.experimental import pallas as pl


def kernel(t, x, edge_index, W_sheaf):
    raise NotImplementedError("write your pallas kernel here")


# speedup vs baseline: 95.8814x; 95.8814x over previous
import dataclasses
import functools

import jax
import jax.numpy as jnp
from jax import lax
from jax.experimental import pallas as pl
from jax.experimental.pallas import tpu as pltpu
from jax.experimental.pallas import tpu_sc as plsc

N = 50000
D = 2
H = 32
E = 800000

AUGW = 48
BLK = 128
NBLK = E // BLK
NS = 16
ROWS_PER_TILE = N // NS
ZR = 125
OCH = 625

_mesh = plsc.VectorSubcoreMesh(core_axis_name="c", subcore_axis_name="s")

_sc_params = pltpu.CompilerParams()
if "needs_layout_passes" in pltpu.CompilerParams.__dataclass_fields__:
    _sc_params = dataclasses.replace(_sc_params, needs_layout_passes=False)
if "use_tc_tiling_on_sc" in pltpu.CompilerParams.__dataclass_fields__:
    _sc_params = dataclasses.replace(_sc_params, use_tc_tiling_on_sc=False)


def _aug_body(x_ref, w_ref, out_ref):
    xb = x_ref[...]
    w = w_ref[...]
    xe = xb[:, 0, :]
    xo = xb[:, 1, :]
    f32 = jnp.float32
    a = (jnp.dot(xe, w[0:32], preferred_element_type=f32)
         + jnp.dot(xo, w[32:64], preferred_element_type=f32))
    b = (jnp.dot(xe, w[64:96], preferred_element_type=f32)
         + jnp.dot(xo, w[96:128], preferred_element_type=f32))
    nb = xb.shape[0]
    out_ref[...] = jnp.concatenate(
        [xb, a[:, :, None], b[:, :, None],
         jnp.zeros((nb, 2, AUGW - 34), f32)], axis=2)


def _build_aug(x3, w):
    nb = 1000
    return pl.pallas_call(
        _aug_body,
        grid=(N // nb,),
        in_specs=[
            pl.BlockSpec((nb, 2, H), lambda i: (i, 0, 0)),
            pl.BlockSpec((2 * D * H, D), lambda i: (0, 0)),
        ],
        out_specs=pl.BlockSpec((nb, 2, AUGW), lambda i: (i, 0, 0)),
        out_shape=jax.ShapeDtypeStruct((N, 2, AUGW), jnp.float32),
    )(x3, w)


def _tanh16(z):
    e = jnp.exp(z * 2.0)
    return 1.0 - 2.0 / (e + 1.0)


@functools.partial(
    pl.kernel,
    out_type=jax.ShapeDtypeStruct((N, D, H), jnp.float32),
    mesh=_mesh,
    compiler_params=_sc_params,
    scratch_types=[
        pltpu.VMEM_SHARED((N, 1, H), jnp.float32),
        pltpu.VMEM((BLK,), jnp.int32),
        pltpu.VMEM((BLK,), jnp.int32),
        pltpu.VMEM((BLK,), jnp.int32),
        pltpu.VMEM((BLK,), jnp.int32),
        pltpu.VMEM((BLK, AUGW), jnp.float32),
        pltpu.VMEM((BLK, AUGW), jnp.float32),
        pltpu.VMEM((BLK, 1, H), jnp.float32),
        pltpu.VMEM((BLK, 1, H), jnp.float32),
        pltpu.VMEM((BLK,), jnp.float32),
        pltpu.VMEM((BLK,), jnp.float32),
        pltpu.VMEM((BLK,), jnp.float32),
        pltpu.VMEM((ZR, 1, H), jnp.float32),
        pltpu.SemaphoreType.DMA,
        pltpu.SemaphoreType.DMA,
    ],
)
def _sheaf_sc(xaug_hbm, row_hbm, col_hbm, out_hbm, acc, rowv, colv, giu, giv,
              xu, xv, pu, pv, pdv, ml2v, mr2v, zb, sem1, sem2):
    cid = lax.axis_index("c")
    sid = lax.axis_index("s")
    d = cid

    @pl.loop(0, ZR)
    def _(i):
        zero16 = jnp.zeros((16,), jnp.float32)
        zb[i, 0, pl.ds(0, 16)] = zero16
        zb[i, 0, pl.ds(16, 16)] = zero16

    @pl.loop(0, ROWS_PER_TILE // ZR)
    def _(k):
        pltpu.sync_copy(zb, acc.at[pl.ds(sid * ROWS_PER_TILE + k * ZR, ZR)])

    plsc.subcore_barrier()

    @pl.loop(sid, NBLK, step=NS)
    def _(blk):
        e0 = blk * BLK
        pltpu.sync_copy(row_hbm.at[pl.ds(e0, BLK)], rowv)
        pltpu.sync_copy(col_hbm.at[pl.ds(e0, BLK)], colv)

        @pl.loop(0, BLK // 16)
        def _(g):
            sl = pl.ds(g * 16, 16)
            giu[sl] = rowv[sl] * 2 + d
            giv[sl] = colv[sl] * 2 + d

        cp1 = pltpu.async_copy(xaug_hbm.at[giu], xu, sem1)
        cp2 = pltpu.async_copy(xaug_hbm.at[giv], xv, sem2)
        cp1.wait()
        cp2.wait()

        @pl.loop(0, BLK // 16)
        def _(g):
            rows = lax.iota(jnp.int32, 16) + g * 16
            ca = jnp.full((16,), 32, jnp.int32)
            cb = jnp.full((16,), 33, jnp.int32)
            au = plsc.load_gather(xu, [rows, ca])
            bu = plsc.load_gather(xu, [rows, cb])
            av = plsc.load_gather(xv, [rows, ca])
            bv = plsc.load_gather(xv, [rows, cb])
            ml = _tanh16(au + bv)
            mr = _tanh16(av + bu)
            sl = pl.ds(g * 16, 16)
            pdv[sl] = ml * mr
            ml2v[sl] = ml * ml
            mr2v[sl] = mr * mr

        @pl.loop(0, BLK)
        def _(e):
            ef = jnp.full((16,), e, jnp.int32)
            cp_ = plsc.load_gather(pdv, [ef])
            cu_ = plsc.load_gather(ml2v, [ef])
            cv_ = plsc.load_gather(mr2v, [ef])
            lo = pl.ds(0, 16)
            hi = pl.ds(16, 16)
            xu0 = xu[e, lo]
            xu1 = xu[e, hi]
            xv0 = xv[e, lo]
            xv1 = xv[e, hi]
            pu[e, 0, lo] = cp_ * xv0 - cu_ * xu0
            pu[e, 0, hi] = cp_ * xv1 - cu_ * xu1
            pv[e, 0, lo] = cp_ * xu0 - cv_ * xv0
            pv[e, 0, hi] = cp_ * xu1 - cv_ * xv1

        pltpu.sync_copy(pu, acc.at[rowv], add=True)
        pltpu.sync_copy(pv, acc.at[colv], add=True)

    plsc.subcore_barrier()

    @pl.loop(0, ROWS_PER_TILE // OCH)
    def _(k):
        base = sid * ROWS_PER_TILE + k * OCH
        pltpu.sync_copy(acc.at[pl.ds(base, OCH)],
                        out_hbm.at[pl.ds(base, OCH), pl.ds(d, 1)])


def kernel(t, x, edge_index, W_sheaf):
    x3 = x.reshape(N, D, H)
    xaug = _build_aug(x3, W_sheaf).reshape(2 * N, AUGW)
    row = edge_index[0]
    col = edge_index[1]
    y = _sheaf_sc(xaug, row, col)
    return y.reshape(N * D, H)
